# baseline (speedup 1.0000x reference)
import functools

import jax
import jax.numpy as jnp
from jax import lax
from jax.experimental import pallas as pl
from jax.experimental.pallas import tpu as pltpu
from jax.experimental.pallas import tpu_sc as plsc

N = 10000
D = 128
HD = D // 2
E = 320000
NC = 2
NS = 16
CHUNK = 40
NCHUNK = 500
NPHASE = 1
PCHUNK = NCHUNK // NPHASE
E_PAD = NS * NCHUNK * CHUNK
ACC_N = 10240
RPT = ACC_N // NS
ZROWS = 40
DEGW = 16
NB = 5
K = 4


def _sc_aggregate(x2, rowx, col):
    mesh = plsc.VectorSubcoreMesh(core_axis_name="c", subcore_axis_name="s")

    @functools.partial(
        pl.kernel,
        out_type=(
            jax.ShapeDtypeStruct((NC * ACC_N, HD), jnp.float32),
            jax.ShapeDtypeStruct((NC * ACC_N, DEGW), jnp.float32),
        ),
        mesh=mesh,
        scratch_types=[
            pltpu.VMEM((PCHUNK, CHUNK), jnp.int32),
            pltpu.VMEM((PCHUNK, CHUNK), jnp.int32),
            pltpu.VMEM((CHUNK, DEGW), jnp.float32),
            pltpu.VMEM((ZROWS, DEGW), jnp.float32),
            pltpu.VMEM_SHARED((ACC_N, HD), jnp.float32),
            pltpu.VMEM_SHARED((ACC_N, DEGW), jnp.float32),
            pltpu.SemaphoreType.DMA,
            pltpu.SemaphoreType.DMA,
            pltpu.SemaphoreType.DMA,
        ]
        + [pltpu.VMEM((CHUNK, HD), jnp.float32) for _ in range(NB)],
        compiler_params=pltpu.CompilerParams(use_tc_tiling_on_sc=False),
    )
    def sc_kernel(x_hbm, rowx_hbm, col_hbm, part_hbm, degp_hbm,
                  row_v, col_v, ones_v, zdeg, acc_s, deg_s,
                  sem_g, sem_s, sem_d, *gbuf):
        c = lax.axis_index("c")
        s = lax.axis_index("s")

        zeros16 = jnp.zeros((16,), jnp.float32)
        ones16 = jnp.ones((16,), jnp.float32)

        @pl.loop(0, ZROWS)
        def _(i):
            zdeg[i, :] = zeros16
            ones_v[i, :] = ones16

            @pl.loop(0, HD // 16)
            def _(k):
                gbuf[0][i, pl.ds(k * 16, 16)] = zeros16

        @pl.loop(0, RPT // ZROWS)
        def _(q):
            base = s * RPT + q * ZROWS
            pltpu.sync_copy(gbuf[0], acc_s.at[pl.ds(base, ZROWS)])
            pltpu.sync_copy(zdeg, deg_s.at[pl.ds(base, ZROWS)])

        plsc.subcore_barrier()

        for h in range(NPHASE):
            pltpu.sync_copy(rowx_hbm.at[c, s, h], row_v)
            pltpu.sync_copy(col_hbm.at[s, h], col_v)

            @pl.loop(0, PCHUNK)
            def _(j):
                pltpu.sync_copy(x_hbm.at[row_v.at[j]], gbuf[0])
                pltpu.sync_copy(gbuf[0], acc_s.at[col_v.at[j]], add=True)

                @pl.when((j % 2) == c)
                def _():
                    pltpu.sync_copy(ones_v, deg_s.at[col_v.at[j]], add=True)

        plsc.subcore_barrier()

        out_base = c * ACC_N + s * RPT
        pltpu.sync_copy(acc_s.at[pl.ds(s * RPT, RPT)],
                        part_hbm.at[pl.ds(out_base, RPT)])
        pltpu.sync_copy(deg_s.at[pl.ds(s * RPT, RPT)],
                        degp_hbm.at[pl.ds(out_base, RPT)])

    return sc_kernel(x2, rowx, col)


def _tc_combine(x, part, degp, W_self, W_neigh, bias2d):
    R = 1000

    def body(x_ref, part_ref, degp_ref, ws_ref, wn_ref, b_ref, o_ref):
        a = jnp.concatenate([part_ref[0], part_ref[1]], axis=1)
        d = degp_ref[0] + degp_ref[1]
        dcol = jnp.maximum(d[:, 0:1], 1.0)
        agg = a / dcol
        o_ref[...] = (
            jnp.dot(x_ref[...], ws_ref[...], preferred_element_type=jnp.float32)
            + jnp.dot(agg, wn_ref[...], preferred_element_type=jnp.float32)
            + b_ref[...]
        )

    return pl.pallas_call(
        body,
        grid=(N // R,),
        in_specs=[
            pl.BlockSpec((R, D), lambda i: (i, 0)),
            pl.BlockSpec((NC, R, HD), lambda i: (0, i, 0)),
            pl.BlockSpec((NC, R, DEGW), lambda i: (0, i, 0)),
            pl.BlockSpec((D, D), lambda i: (0, 0)),
            pl.BlockSpec((D, D), lambda i: (0, 0)),
            pl.BlockSpec((1, D), lambda i: (0, 0)),
        ],
        out_specs=pl.BlockSpec((R, D), lambda i: (i, 0)),
        out_shape=jax.ShapeDtypeStruct((N, D), jnp.float32),
    )(x, part, degp, W_self, W_neigh, bias2d)


def kernel(x, edge_index, W_self, W_neigh, bias):
    ei = edge_index.astype(jnp.int32)
    pad = E_PAD - E
    row = jnp.concatenate([ei[0], jnp.zeros((pad,), jnp.int32)])
    pad_cols = N + (jnp.arange(pad, dtype=jnp.int32) % (ACC_N - N))
    col = jnp.concatenate([ei[1], pad_cols])
    row2 = 2 * row
    rowx = jnp.stack([row2, row2 + 1])
    rowx = rowx.reshape(NC, NS, NPHASE, PCHUNK, CHUNK)
    col = col.reshape(NS, NPHASE, PCHUNK, CHUNK)
    x2 = x.reshape(2 * N, HD)
    part, degp = _sc_aggregate(x2, rowx, col)
    part = part.reshape(NC, ACC_N, HD)
    degp = degp.reshape(NC, ACC_N, DEGW)
    return _tc_combine(x, part, degp, W_self, W_neigh, bias.reshape(1, D))


# speedup vs baseline: 1.4823x; 1.4823x over previous
import functools

import jax
import jax.numpy as jnp
from jax import lax
from jax.experimental import pallas as pl
from jax.experimental.pallas import tpu as pltpu
from jax.experimental.pallas import tpu_sc as plsc

N = 10000
D = 128
HD = D // 2
E = 320000
NC = 2
NS = 16
CHUNK = 80
NCHUNK = 250
NPHASE = 1
PCHUNK = NCHUNK // NPHASE
E_PAD = NS * NCHUNK * CHUNK
ACC_N = 10240
RPT = ACC_N // NS
ZROWS = 80
DEGW = 16
NB = 5
K = 4


def _sc_aggregate(x2, rowx, col):
    mesh = plsc.VectorSubcoreMesh(core_axis_name="c", subcore_axis_name="s")

    @functools.partial(
        pl.kernel,
        out_type=(
            jax.ShapeDtypeStruct((NC * ACC_N, HD), jnp.float32),
            jax.ShapeDtypeStruct((NC * ACC_N, DEGW), jnp.float32),
        ),
        mesh=mesh,
        scratch_types=[
            pltpu.VMEM((PCHUNK, CHUNK), jnp.int32),
            pltpu.VMEM((PCHUNK, CHUNK), jnp.int32),
            pltpu.VMEM((CHUNK, DEGW), jnp.float32),
            pltpu.VMEM((ZROWS, DEGW), jnp.float32),
            pltpu.VMEM_SHARED((ACC_N, HD), jnp.float32),
            pltpu.VMEM_SHARED((ACC_N, DEGW), jnp.float32),
            pltpu.SemaphoreType.DMA,
            pltpu.SemaphoreType.DMA,
            pltpu.SemaphoreType.DMA,
        ]
        + [pltpu.VMEM((CHUNK, HD), jnp.float32) for _ in range(NB)],
        compiler_params=pltpu.CompilerParams(use_tc_tiling_on_sc=False),
    )
    def sc_kernel(x_hbm, rowx_hbm, col_hbm, part_hbm, degp_hbm,
                  row_v, col_v, ones_v, zdeg, acc_s, deg_s,
                  sem_g, sem_s, sem_d, *gbuf):
        c = lax.axis_index("c")
        s = lax.axis_index("s")

        zeros16 = jnp.zeros((16,), jnp.float32)
        ones16 = jnp.ones((16,), jnp.float32)

        @pl.loop(0, ZROWS)
        def _(i):
            zdeg[i, :] = zeros16
            ones_v[i, :] = ones16

            @pl.loop(0, HD // 16)
            def _(k):
                gbuf[0][i, pl.ds(k * 16, 16)] = zeros16

        @pl.loop(0, RPT // ZROWS)
        def _(q):
            base = s * RPT + q * ZROWS
            pltpu.sync_copy(gbuf[0], acc_s.at[pl.ds(base, ZROWS)])
            pltpu.sync_copy(zdeg, deg_s.at[pl.ds(base, ZROWS)])

        plsc.subcore_barrier()

        for h in range(NPHASE):
            pltpu.sync_copy(rowx_hbm.at[c, s, h], row_v)
            pltpu.sync_copy(col_hbm.at[s, h], col_v)

            @pl.loop(0, PCHUNK)
            def _(j):
                pltpu.sync_copy(x_hbm.at[row_v.at[j]], gbuf[0])
                pltpu.sync_copy(gbuf[0], acc_s.at[col_v.at[j]], add=True)

            dlo = c * (PCHUNK // 2)

            @pl.loop(0, PCHUNK // 2)
            def _(j):
                pltpu.sync_copy(ones_v, deg_s.at[col_v.at[dlo + j]], add=True)

        plsc.subcore_barrier()

        out_base = c * ACC_N + s * RPT
        pltpu.sync_copy(acc_s.at[pl.ds(s * RPT, RPT)],
                        part_hbm.at[pl.ds(out_base, RPT)])
        pltpu.sync_copy(deg_s.at[pl.ds(s * RPT, RPT)],
                        degp_hbm.at[pl.ds(out_base, RPT)])

    return sc_kernel(x2, rowx, col)


def _tc_combine(x, part, degp, W_self, W_neigh, bias2d):
    R = 1000

    def body(x_ref, part_ref, degp_ref, ws_ref, wn_ref, b_ref, o_ref):
        a = jnp.concatenate([part_ref[0], part_ref[1]], axis=1)
        d = degp_ref[0] + degp_ref[1]
        dcol = jnp.maximum(d[:, 0:1], 1.0)
        agg = a / dcol
        o_ref[...] = (
            jnp.dot(x_ref[...], ws_ref[...], preferred_element_type=jnp.float32)
            + jnp.dot(agg, wn_ref[...], preferred_element_type=jnp.float32)
            + b_ref[...]
        )

    return pl.pallas_call(
        body,
        grid=(N // R,),
        in_specs=[
            pl.BlockSpec((R, D), lambda i: (i, 0)),
            pl.BlockSpec((NC, R, HD), lambda i: (0, i, 0)),
            pl.BlockSpec((NC, R, DEGW), lambda i: (0, i, 0)),
            pl.BlockSpec((D, D), lambda i: (0, 0)),
            pl.BlockSpec((D, D), lambda i: (0, 0)),
            pl.BlockSpec((1, D), lambda i: (0, 0)),
        ],
        out_specs=pl.BlockSpec((R, D), lambda i: (i, 0)),
        out_shape=jax.ShapeDtypeStruct((N, D), jnp.float32),
    )(x, part, degp, W_self, W_neigh, bias2d)


def kernel(x, edge_index, W_self, W_neigh, bias):
    ei = edge_index.astype(jnp.int32)
    pad = E_PAD - E
    row = jnp.concatenate([ei[0], jnp.zeros((pad,), jnp.int32)])
    pad_cols = N + (jnp.arange(pad, dtype=jnp.int32) % (ACC_N - N))
    col = jnp.concatenate([ei[1], pad_cols])
    row2 = 2 * row
    rowx = jnp.stack([row2, row2 + 1])
    rowx = rowx.reshape(NC, NS, NPHASE, PCHUNK, CHUNK)
    col = col.reshape(NS, NPHASE, PCHUNK, CHUNK)
    x2 = x.reshape(2 * N, HD)
    part, degp = _sc_aggregate(x2, rowx, col)
    part = part.reshape(NC, ACC_N, HD)
    degp = degp.reshape(NC, ACC_N, DEGW)
    return _tc_combine(x, part, degp, W_self, W_neigh, bias.reshape(1, D))


# speedup vs baseline: 3.1571x; 2.1299x over previous
import functools

import jax
import jax.numpy as jnp
from jax import lax
from jax.experimental import pallas as pl
from jax.experimental.pallas import tpu as pltpu
from jax.experimental.pallas import tpu_sc as plsc

N = 10000
D = 128
HD = D // 2
E = 320000
NC = 2
NS = 16
CHUNK = 80
NCHUNK = 250
NPHASE = 1
PCHUNK = NCHUNK // NPHASE
E_PAD = NS * NCHUNK * CHUNK
ACC_N = 10240
RPT = ACC_N // NS
ZROWS = 80
DEGW = 16
NB = 6
K = 4
MAIN_LO = NB
MAIN_HI = NCHUNK - K


def _sc_aggregate(x2, rowx, col):
    mesh = plsc.VectorSubcoreMesh(core_axis_name="c", subcore_axis_name="s")

    @functools.partial(
        pl.kernel,
        out_type=(
            jax.ShapeDtypeStruct((NC * ACC_N, HD), jnp.float32),
            jax.ShapeDtypeStruct((NC * ACC_N, DEGW), jnp.float32),
        ),
        mesh=mesh,
        scratch_types=[
            pltpu.VMEM((PCHUNK, CHUNK), jnp.int32),
            pltpu.VMEM((PCHUNK, CHUNK), jnp.int32),
            pltpu.VMEM((CHUNK, DEGW), jnp.float32),
            pltpu.VMEM((ZROWS, DEGW), jnp.float32),
            pltpu.VMEM_SHARED((ACC_N, HD), jnp.float32),
            pltpu.VMEM_SHARED((ACC_N, DEGW), jnp.float32),
            pltpu.SemaphoreType.DMA,
            pltpu.SemaphoreType.DMA,
            pltpu.SemaphoreType.DMA,
        ]
        + [pltpu.VMEM((CHUNK, HD), jnp.float32) for _ in range(NB)],
        compiler_params=pltpu.CompilerParams(use_tc_tiling_on_sc=False),
    )
    def sc_kernel(x_hbm, rowx_hbm, col_hbm, part_hbm, degp_hbm,
                  row_v, col_v, ones_v, zdeg, acc_s, deg_s,
                  sem_g, sem_s, sem_d, *gbuf):
        c = lax.axis_index("c")
        s = lax.axis_index("s")

        zeros16 = jnp.zeros((16,), jnp.float32)
        ones16 = jnp.ones((16,), jnp.float32)

        @pl.loop(0, ZROWS)
        def _(i):
            zdeg[i, :] = zeros16
            ones_v[i, :] = ones16

            @pl.loop(0, HD // 16)
            def _(k):
                gbuf[0][i, pl.ds(k * 16, 16)] = zeros16

        @pl.loop(0, RPT // ZROWS)
        def _(q):
            base = s * RPT + q * ZROWS
            pltpu.sync_copy(gbuf[0], acc_s.at[pl.ds(base, ZROWS)])
            pltpu.sync_copy(zdeg, deg_s.at[pl.ds(base, ZROWS)])

        plsc.subcore_barrier()

        for h in range(NPHASE):
            pltpu.sync_copy(rowx_hbm.at[c, s, h], row_v)
            pltpu.sync_copy(col_hbm.at[s, h], col_v)

            def gwait(j, b):
                pltpu.make_async_copy(
                    x_hbm.at[row_v.at[j]], gbuf[b], sem_g).wait()

            def gfire(j, b):
                pltpu.async_copy(x_hbm.at[row_v.at[j]], gbuf[b], sem_g)

            def sfire(j, b):
                pltpu.async_copy(gbuf[b], acc_s.at[col_v.at[j]], sem_s,
                                 add=True)

            def sdrain():
                pltpu.make_async_copy(
                    gbuf[0], acc_s.at[col_v.at[0]], sem_s).wait()

            for b in range(K):
                gfire(b, b)

            for j in range(NB):
                gwait(j, j % NB)
                sfire(j, j % NB)
                if j >= NB - K:
                    sdrain()
                gfire(j + K, (j + K) % NB)

            @pl.loop(MAIN_LO, MAIN_HI, step=NB)
            def _(oj):
                for b in range(NB):
                    j = oj + b
                    gwait(j, b)
                    sfire(j, b)
                    sdrain()
                    pltpu.async_copy(
                        x_hbm.at[row_v.at[j + K]], gbuf[(b + K) % NB], sem_g)

            for jj in range(MAIN_HI, PCHUNK):
                gwait(jj, jj % NB)
                sfire(jj, jj % NB)

            @pl.loop(0, NB)
            def _(_):
                sdrain()

            dlo = c * (PCHUNK // 2)

            @pl.loop(0, PCHUNK // 2)
            def _(j):
                pltpu.async_copy(ones_v, deg_s.at[col_v.at[dlo + j]], sem_d,
                                 add=True)

            @pl.loop(0, PCHUNK // 2)
            def _(_):
                pltpu.make_async_copy(
                    ones_v, deg_s.at[col_v.at[0]], sem_d).wait()

        plsc.subcore_barrier()

        out_base = c * ACC_N + s * RPT
        pltpu.sync_copy(acc_s.at[pl.ds(s * RPT, RPT)],
                        part_hbm.at[pl.ds(out_base, RPT)])
        pltpu.sync_copy(deg_s.at[pl.ds(s * RPT, RPT)],
                        degp_hbm.at[pl.ds(out_base, RPT)])

    return sc_kernel(x2, rowx, col)


def _tc_combine(x, part, degp, W_self, W_neigh, bias2d):
    R = 1000

    def body(x_ref, part_ref, degp_ref, ws_ref, wn_ref, b_ref, o_ref):
        a = jnp.concatenate([part_ref[0], part_ref[1]], axis=1)
        d = degp_ref[0] + degp_ref[1]
        dcol = jnp.maximum(d[:, 0:1], 1.0)
        agg = a / dcol
        o_ref[...] = (
            jnp.dot(x_ref[...], ws_ref[...], preferred_element_type=jnp.float32)
            + jnp.dot(agg, wn_ref[...], preferred_element_type=jnp.float32)
            + b_ref[...]
        )

    return pl.pallas_call(
        body,
        grid=(N // R,),
        in_specs=[
            pl.BlockSpec((R, D), lambda i: (i, 0)),
            pl.BlockSpec((NC, R, HD), lambda i: (0, i, 0)),
            pl.BlockSpec((NC, R, DEGW), lambda i: (0, i, 0)),
            pl.BlockSpec((D, D), lambda i: (0, 0)),
            pl.BlockSpec((D, D), lambda i: (0, 0)),
            pl.BlockSpec((1, D), lambda i: (0, 0)),
        ],
        out_specs=pl.BlockSpec((R, D), lambda i: (i, 0)),
        out_shape=jax.ShapeDtypeStruct((N, D), jnp.float32),
    )(x, part, degp, W_self, W_neigh, bias2d)


def kernel(x, edge_index, W_self, W_neigh, bias):
    ei = edge_index.astype(jnp.int32)
    pad = E_PAD - E
    row = jnp.concatenate([ei[0], jnp.zeros((pad,), jnp.int32)])
    pad_cols = N + (jnp.arange(pad, dtype=jnp.int32) % (ACC_N - N))
    col = jnp.concatenate([ei[1], pad_cols])
    row2 = 2 * row
    rowx = jnp.stack([row2, row2 + 1])
    rowx = rowx.reshape(NC, NS, NPHASE, PCHUNK, CHUNK)
    col = col.reshape(NS, NPHASE, PCHUNK, CHUNK)
    x2 = x.reshape(2 * N, HD)
    part, degp = _sc_aggregate(x2, rowx, col)
    part = part.reshape(NC, ACC_N, HD)
    degp = degp.reshape(NC, ACC_N, DEGW)
    return _tc_combine(x, part, degp, W_self, W_neigh, bias.reshape(1, D))


# speedup vs baseline: 3.5789x; 1.1336x over previous
import functools

import jax
import jax.numpy as jnp
from jax import lax
from jax.experimental import pallas as pl
from jax.experimental.pallas import tpu as pltpu
from jax.experimental.pallas import tpu_sc as plsc

N = 10000
D = 128
HD = D // 2
E = 320000
NC = 2
NS = 16
CHUNK = 80
NCHUNK = 250
ACC_N = 10240
RPT = ACC_N // NS
ZROWS = 80
DEGW = 16
NB = 6
K = 4
MAIN_LO = NB
MAIN_HI = NCHUNK - K


def _sc_aggregate(x2, row, col):
    mesh = plsc.VectorSubcoreMesh(core_axis_name="c", subcore_axis_name="s")

    @functools.partial(
        pl.kernel,
        out_type=(
            jax.ShapeDtypeStruct((NC * ACC_N, HD), jnp.float32),
            jax.ShapeDtypeStruct((NC * ACC_N, DEGW), jnp.float32),
        ),
        mesh=mesh,
        scratch_types=[
            pltpu.VMEM((NCHUNK, CHUNK), jnp.int32),
            pltpu.VMEM((NCHUNK, CHUNK), jnp.int32),
            pltpu.VMEM((CHUNK, DEGW), jnp.float32),
            pltpu.VMEM((ZROWS, DEGW), jnp.float32),
            pltpu.VMEM_SHARED((ACC_N, HD), jnp.float32),
            pltpu.VMEM_SHARED((ACC_N, DEGW), jnp.float32),
            pltpu.SemaphoreType.DMA,
            pltpu.SemaphoreType.DMA,
            pltpu.SemaphoreType.DMA,
        ]
        + [pltpu.VMEM((CHUNK, HD), jnp.float32) for _ in range(NB)],
        compiler_params=pltpu.CompilerParams(use_tc_tiling_on_sc=False),
    )
    def sc_kernel(x_hbm, row_hbm, col_hbm, part_hbm, degp_hbm,
                  row_v, col_v, ones_v, zdeg, acc_s, deg_s,
                  sem_g, sem_s, sem_d, *gbuf):
        c = lax.axis_index("c")
        s = lax.axis_index("s")

        zeros16 = jnp.zeros((16,), jnp.float32)
        ones16 = jnp.ones((16,), jnp.float32)
        ctile = jnp.full((16,), 0, jnp.int32) + c

        @pl.loop(0, ZROWS)
        def _(i):
            zdeg[i, :] = zeros16
            ones_v[i, :] = ones16

            @pl.loop(0, HD // 16)
            def _(k):
                gbuf[0][i, pl.ds(k * 16, 16)] = zeros16

        @pl.loop(0, RPT // ZROWS)
        def _(q):
            base = s * RPT + q * ZROWS
            pltpu.async_copy(gbuf[0], acc_s.at[pl.ds(base, ZROWS)], sem_s)
            pltpu.async_copy(zdeg, deg_s.at[pl.ds(base, ZROWS)], sem_s)

        pltpu.async_copy(row_hbm.at[s], row_v, sem_g)
        pltpu.async_copy(col_hbm.at[s], col_v, sem_g)
        pltpu.make_async_copy(row_hbm.at[s], row_v, sem_g).wait()
        pltpu.make_async_copy(col_hbm.at[s], col_v, sem_g).wait()

        def rewrite(j):
            for k in range(CHUNK // 16):
                v = row_v[j, pl.ds(k * 16, 16)]
                row_v[j, pl.ds(k * 16, 16)] = v * 2 + ctile

        def gwait(j, b):
            pltpu.make_async_copy(
                x_hbm.at[row_v.at[j]], gbuf[b], sem_g).wait()

        def gfire(j, b):
            pltpu.async_copy(x_hbm.at[row_v.at[j]], gbuf[b], sem_g)

        def sfire(j, b):
            pltpu.async_copy(gbuf[b], acc_s.at[col_v.at[j]], sem_s, add=True)

        def sdrain():
            pltpu.make_async_copy(
                gbuf[0], acc_s.at[col_v.at[0]], sem_s).wait()

        for j in range(K):
            rewrite(j)

        @pl.loop(0, RPT // ZROWS)
        def _(_):
            pltpu.make_async_copy(gbuf[0], acc_s.at[pl.ds(0, ZROWS)],
                                  sem_s).wait()
            pltpu.make_async_copy(zdeg, deg_s.at[pl.ds(0, ZROWS)],
                                  sem_s).wait()

        for b in range(K):
            gfire(b, b)

        plsc.subcore_barrier()

        for j in range(NB):
            rewrite(j + K)
            gwait(j, j % NB)
            sfire(j, j % NB)
            if j >= NB - K:
                sdrain()
            gfire(j + K, (j + K) % NB)

        @pl.loop(MAIN_LO, MAIN_HI, step=NB)
        def _(oj):
            for b in range(NB):
                j = oj + b
                rewrite(j + K)
                gwait(j, b)
                sfire(j, b)
                sdrain()
                pltpu.async_copy(
                    x_hbm.at[row_v.at[j + K]], gbuf[(b + K) % NB], sem_g)

        for jj in range(MAIN_HI, NCHUNK):
            gwait(jj, jj % NB)
            sfire(jj, jj % NB)

        @pl.loop(0, NB)
        def _(_):
            sdrain()

        dlo = c * (NCHUNK // 2)

        @pl.loop(0, NCHUNK // 2)
        def _(j):
            pltpu.async_copy(ones_v, deg_s.at[col_v.at[dlo + j]], sem_d,
                             add=True)

        @pl.loop(0, NCHUNK // 2)
        def _(_):
            pltpu.make_async_copy(
                ones_v, deg_s.at[col_v.at[0]], sem_d).wait()

        plsc.subcore_barrier()

        out_base = c * ACC_N + s * RPT
        pltpu.sync_copy(acc_s.at[pl.ds(s * RPT, RPT)],
                        part_hbm.at[pl.ds(out_base, RPT)])
        pltpu.sync_copy(deg_s.at[pl.ds(s * RPT, RPT)],
                        degp_hbm.at[pl.ds(out_base, RPT)])

    return sc_kernel(x2, row, col)


def _tc_combine(x, part, degp, W_self, W_neigh, bias2d):
    R = 1000

    def body(x_ref, part_ref, degp_ref, ws_ref, wn_ref, b_ref, o_ref):
        a = jnp.concatenate([part_ref[0], part_ref[1]], axis=1)
        d = degp_ref[0] + degp_ref[1]
        dcol = jnp.maximum(d[:, 0:1], 1.0)
        agg = a / dcol
        o_ref[...] = (
            jnp.dot(x_ref[...], ws_ref[...], preferred_element_type=jnp.float32)
            + jnp.dot(agg, wn_ref[...], preferred_element_type=jnp.float32)
            + b_ref[...]
        )

    return pl.pallas_call(
        body,
        grid=(N // R,),
        in_specs=[
            pl.BlockSpec((R, D), lambda i: (i, 0)),
            pl.BlockSpec((NC, R, HD), lambda i: (0, i, 0)),
            pl.BlockSpec((NC, R, DEGW), lambda i: (0, i, 0)),
            pl.BlockSpec((D, D), lambda i: (0, 0)),
            pl.BlockSpec((D, D), lambda i: (0, 0)),
            pl.BlockSpec((1, D), lambda i: (0, 0)),
        ],
        out_specs=pl.BlockSpec((R, D), lambda i: (i, 0)),
        out_shape=jax.ShapeDtypeStruct((N, D), jnp.float32),
    )(x, part, degp, W_self, W_neigh, bias2d)


def kernel(x, edge_index, W_self, W_neigh, bias):
    ei = edge_index.astype(jnp.int32)
    row = ei[0].reshape(NS, NCHUNK, CHUNK)
    col = ei[1].reshape(NS, NCHUNK, CHUNK)
    x2 = x.reshape(2 * N, HD)
    part, degp = _sc_aggregate(x2, row, col)
    part = part.reshape(NC, ACC_N, HD)
    degp = degp.reshape(NC, ACC_N, DEGW)
    return _tc_combine(x, part, degp, W_self, W_neigh, bias.reshape(1, D))


# speedup vs baseline: 4.0621x; 1.1350x over previous
import functools

import jax
import jax.numpy as jnp
from jax import lax
from jax.experimental import pallas as pl
from jax.experimental.pallas import tpu as pltpu
from jax.experimental.pallas import tpu_sc as plsc

N = 10000
D = 128
HD = D // 2
E = 320000
NC = 2
NS = 16
CHUNK = 80
NCHUNK = 250
ACC_N = 10240
RPT = ACC_N // NS
ZROWS = 80
DEGW = 16
NB = 6
K = 4
MAIN_LO = NB
MAIN_HI = NCHUNK - K


def _sc_aggregate(x2, ei):
    mesh = plsc.VectorSubcoreMesh(core_axis_name="c", subcore_axis_name="s")

    @functools.partial(
        pl.kernel,
        out_type=(
            jax.ShapeDtypeStruct((ACC_N, D), jnp.float32),
            jax.ShapeDtypeStruct((NC, ACC_N, DEGW), jnp.float32),
        ),
        mesh=mesh,
        scratch_types=[
            pltpu.VMEM((NCHUNK, CHUNK), jnp.int32),
            pltpu.VMEM((NCHUNK, CHUNK), jnp.int32),
            pltpu.VMEM((CHUNK, DEGW), jnp.float32),
            pltpu.VMEM((ZROWS, DEGW), jnp.float32),
            pltpu.VMEM_SHARED((ACC_N, HD), jnp.float32),
            pltpu.VMEM_SHARED((ACC_N, DEGW), jnp.float32),
            pltpu.SemaphoreType.DMA,
            pltpu.SemaphoreType.DMA,
            pltpu.SemaphoreType.DMA,
        ]
        + [pltpu.VMEM((CHUNK, HD), jnp.float32) for _ in range(NB)],
        compiler_params=pltpu.CompilerParams(use_tc_tiling_on_sc=False),
    )
    def sc_kernel(x_hbm, ei_hbm, part_hbm, degp_hbm,
                  row_v, col_v, ones_v, zdeg, acc_s, deg_s,
                  sem_g, sem_s, sem_d, *gbuf):
        c = lax.axis_index("c")
        s = lax.axis_index("s")

        zeros16 = jnp.zeros((16,), jnp.float32)
        ones16 = jnp.ones((16,), jnp.float32)
        ctile = jnp.full((16,), 0, jnp.int32) + c

        @pl.loop(0, ZROWS)
        def _(i):
            zdeg[i, :] = zeros16
            ones_v[i, :] = ones16

            @pl.loop(0, HD // 16)
            def _(k):
                gbuf[0][i, pl.ds(k * 16, 16)] = zeros16

        @pl.loop(0, RPT // ZROWS)
        def _(q):
            base = s * RPT + q * ZROWS
            pltpu.async_copy(gbuf[0], acc_s.at[pl.ds(base, ZROWS)], sem_s)
            pltpu.async_copy(zdeg, deg_s.at[pl.ds(base, ZROWS)], sem_s)

        pltpu.async_copy(ei_hbm.at[0, s], row_v, sem_g)
        pltpu.async_copy(ei_hbm.at[1, s], col_v, sem_g)
        pltpu.make_async_copy(ei_hbm.at[0, s], row_v, sem_g).wait()
        pltpu.make_async_copy(ei_hbm.at[1, s], col_v, sem_g).wait()

        def rewrite(j):
            for k in range(CHUNK // 16):
                v = row_v[j, pl.ds(k * 16, 16)]
                row_v[j, pl.ds(k * 16, 16)] = v * 2 + ctile

        def gwait(j, b):
            pltpu.make_async_copy(
                x_hbm.at[row_v.at[j]], gbuf[b], sem_g).wait()

        def gfire(j, b):
            pltpu.async_copy(x_hbm.at[row_v.at[j]], gbuf[b], sem_g)

        def sfire(j, b):
            pltpu.async_copy(gbuf[b], acc_s.at[col_v.at[j]], sem_s, add=True)

        def sdrain():
            pltpu.make_async_copy(
                gbuf[0], acc_s.at[col_v.at[0]], sem_s).wait()

        for j in range(K):
            rewrite(j)

        @pl.loop(0, RPT // ZROWS)
        def _(_):
            pltpu.make_async_copy(gbuf[0], acc_s.at[pl.ds(0, ZROWS)],
                                  sem_s).wait()
            pltpu.make_async_copy(zdeg, deg_s.at[pl.ds(0, ZROWS)],
                                  sem_s).wait()

        for b in range(K):
            gfire(b, b)

        plsc.subcore_barrier()

        for j in range(NB):
            rewrite(j + K)
            gwait(j, j % NB)
            sfire(j, j % NB)
            if j >= NB - K:
                sdrain()
            gfire(j + K, (j + K) % NB)

        @pl.loop(MAIN_LO, MAIN_HI, step=NB)
        def _(oj):
            for b in range(NB):
                j = oj + b
                rewrite(j + K)
                gwait(j, b)
                sfire(j, b)
                sdrain()
                pltpu.async_copy(
                    x_hbm.at[row_v.at[j + K]], gbuf[(b + K) % NB], sem_g)

        for jj in range(MAIN_HI, NCHUNK):
            gwait(jj, jj % NB)
            sfire(jj, jj % NB)

        @pl.loop(0, NB)
        def _(_):
            sdrain()

        dlo = c * (NCHUNK // 2)

        @pl.loop(0, NCHUNK // 2)
        def _(j):
            pltpu.async_copy(ones_v, deg_s.at[col_v.at[dlo + j]], sem_d,
                             add=True)

        @pl.loop(0, NCHUNK // 2)
        def _(_):
            pltpu.make_async_copy(
                ones_v, deg_s.at[col_v.at[0]], sem_d).wait()

        plsc.subcore_barrier()

        pltpu.sync_copy(acc_s.at[pl.ds(s * RPT, RPT)],
                        part_hbm.at[pl.ds(s * RPT, RPT),
                                    pl.ds(c * HD, HD)])
        pltpu.sync_copy(deg_s.at[pl.ds(s * RPT, RPT)],
                        degp_hbm.at[c, pl.ds(s * RPT, RPT)])

    return sc_kernel(x2, ei)


def _tc_combine(x, part, degp, W_self, W_neigh, bias2d):
    R = 1000

    def body(x_ref, part_ref, degp_ref, ws_ref, wn_ref, b_ref, o_ref):
        d = degp_ref[0] + degp_ref[1]
        dcol = jnp.maximum(d[:, 0:1], 1.0)
        agg = part_ref[...] / dcol
        o_ref[...] = (
            jnp.dot(x_ref[...], ws_ref[...], preferred_element_type=jnp.float32)
            + jnp.dot(agg, wn_ref[...], preferred_element_type=jnp.float32)
            + b_ref[...]
        )

    return pl.pallas_call(
        body,
        grid=(N // R,),
        in_specs=[
            pl.BlockSpec((R, D), lambda i: (i, 0)),
            pl.BlockSpec((R, D), lambda i: (i, 0)),
            pl.BlockSpec((NC, R, DEGW), lambda i: (0, i, 0)),
            pl.BlockSpec((D, D), lambda i: (0, 0)),
            pl.BlockSpec((D, D), lambda i: (0, 0)),
            pl.BlockSpec((1, D), lambda i: (0, 0)),
        ],
        out_specs=pl.BlockSpec((R, D), lambda i: (i, 0)),
        out_shape=jax.ShapeDtypeStruct((N, D), jnp.float32),
    )(x, part, degp, W_self, W_neigh, bias2d)


def kernel(x, edge_index, W_self, W_neigh, bias):
    ei = edge_index.astype(jnp.int32).reshape(2, NS, NCHUNK, CHUNK)
    x2 = x.reshape(2 * N, HD)
    part, degp = _sc_aggregate(x2, ei)
    return _tc_combine(x, part, degp, W_self, W_neigh, bias.reshape(1, D))


# speedup vs baseline: 4.1400x; 1.0192x over previous
import functools

import jax
import jax.numpy as jnp
from jax import lax
from jax.experimental import pallas as pl
from jax.experimental.pallas import tpu as pltpu
from jax.experimental.pallas import tpu_sc as plsc

N = 10000
D = 128
HD = D // 2
E = 320000
NC = 2
NS = 16
CHUNK = 80
NCHUNK = 250
ACC_N = 10240
RPT = ACC_N // NS
ZROWS = 80
DEGW = 16
NB = 6
K = 4
MAIN_LO = NB
MAIN_HI = NCHUNK - K


def _sc_aggregate(x2, ei):
    mesh = plsc.VectorSubcoreMesh(core_axis_name="c", subcore_axis_name="s")

    @functools.partial(
        pl.kernel,
        out_type=(
            jax.ShapeDtypeStruct((ACC_N, D), jnp.float32),
            jax.ShapeDtypeStruct((NC, ACC_N, DEGW), jnp.float32),
        ),
        mesh=mesh,
        scratch_types=[
            pltpu.VMEM((NCHUNK, CHUNK), jnp.int32),
            pltpu.VMEM((NCHUNK, CHUNK), jnp.int32),
            pltpu.VMEM((CHUNK, DEGW), jnp.float32),
            pltpu.VMEM((ZROWS, DEGW), jnp.float32),
            pltpu.VMEM_SHARED((ACC_N, HD), jnp.float32),
            pltpu.VMEM_SHARED((ACC_N, DEGW), jnp.float32),
            pltpu.SemaphoreType.DMA,
            pltpu.SemaphoreType.DMA,
            pltpu.SemaphoreType.DMA,
        ]
        + [pltpu.VMEM((CHUNK, HD), jnp.float32) for _ in range(NB)],
        compiler_params=pltpu.CompilerParams(use_tc_tiling_on_sc=False),
    )
    def sc_kernel(x_hbm, ei_hbm, part_hbm, degp_hbm,
                  row_v, col_v, ones_v, zdeg, acc_s, deg_s,
                  sem_g, sem_s, sem_d, *gbuf):
        c = lax.axis_index("c")
        s = lax.axis_index("s")

        zeros16 = jnp.zeros((16,), jnp.float32)
        ones16 = jnp.ones((16,), jnp.float32)
        ctile = jnp.full((16,), 0, jnp.int32) + c

        @pl.loop(0, ZROWS)
        def _(i):
            zdeg[i, :] = zeros16
            ones_v[i, :] = ones16

            @pl.loop(0, HD // 16)
            def _(k):
                gbuf[0][i, pl.ds(k * 16, 16)] = zeros16

        @pl.loop(0, RPT // ZROWS)
        def _(q):
            base = s * RPT + q * ZROWS
            pltpu.async_copy(gbuf[0], acc_s.at[pl.ds(base, ZROWS)], sem_s)
            pltpu.async_copy(zdeg, deg_s.at[pl.ds(base, ZROWS)], sem_s)

        pltpu.async_copy(ei_hbm.at[0, s], row_v, sem_g)
        pltpu.async_copy(ei_hbm.at[1, s], col_v, sem_g)
        pltpu.make_async_copy(ei_hbm.at[0, s], row_v, sem_g).wait()
        pltpu.make_async_copy(ei_hbm.at[1, s], col_v, sem_g).wait()

        def rewrite(j):
            for k in range(CHUNK // 16):
                v = row_v[j, pl.ds(k * 16, 16)]
                row_v[j, pl.ds(k * 16, 16)] = v * 2 + ctile

        def gwait(j, b):
            pltpu.make_async_copy(
                x_hbm.at[row_v.at[j]], gbuf[b], sem_g).wait()

        def gfire(j, b):
            pltpu.async_copy(x_hbm.at[row_v.at[j]], gbuf[b], sem_g)

        def sfire(j, b):
            pltpu.async_copy(gbuf[b], acc_s.at[col_v.at[j]], sem_s, add=True)

        def sdrain():
            pltpu.make_async_copy(
                gbuf[0], acc_s.at[col_v.at[0]], sem_s).wait()

        for j in range(K):
            rewrite(j)

        @pl.loop(0, RPT // ZROWS)
        def _(_):
            pltpu.make_async_copy(gbuf[0], acc_s.at[pl.ds(0, ZROWS)],
                                  sem_s).wait()
            pltpu.make_async_copy(zdeg, deg_s.at[pl.ds(0, ZROWS)],
                                  sem_s).wait()

        for b in range(K):
            gfire(b, b)

        plsc.subcore_barrier()

        for j in range(NB):
            rewrite(j + K)
            gwait(j, j % NB)
            sfire(j, j % NB)
            if j >= NB - K:
                sdrain()
            gfire(j + K, (j + K) % NB)

        @pl.loop(MAIN_LO, MAIN_HI, step=NB)
        def _(oj):
            for b in range(NB):
                j = oj + b
                rewrite(j + K)
                gwait(j, b)
                sfire(j, b)
                sdrain()
                pltpu.async_copy(
                    x_hbm.at[row_v.at[j + K]], gbuf[(b + K) % NB], sem_g)

        for jj in range(MAIN_HI, NCHUNK):
            gwait(jj, jj % NB)
            sfire(jj, jj % NB)

        @pl.loop(0, NB)
        def _(_):
            sdrain()

        dlo = c * (NCHUNK // 2)

        @pl.loop(0, NCHUNK // 2)
        def _(j):
            pltpu.async_copy(ones_v, deg_s.at[col_v.at[dlo + j]], sem_d,
                             add=True)

        @pl.loop(0, NCHUNK // 2)
        def _(_):
            pltpu.make_async_copy(
                ones_v, deg_s.at[col_v.at[0]], sem_d).wait()

        plsc.subcore_barrier()

        pltpu.sync_copy(acc_s.at[pl.ds(s * RPT, RPT)],
                        part_hbm.at[pl.ds(s * RPT, RPT),
                                    pl.ds(c * HD, HD)])
        pltpu.sync_copy(deg_s.at[pl.ds(s * RPT, RPT)],
                        degp_hbm.at[c, pl.ds(s * RPT, RPT)])

    return sc_kernel(x2, ei)


def _tc_combine(x, part, degp, W_self, W_neigh, bias2d):
    R = 2000

    def body(x_ref, part_ref, degp_ref, ws_ref, wn_ref, b_ref, o_ref):
        d = degp_ref[0] + degp_ref[1]
        dcol = jnp.maximum(d[:, 0:1], 1.0)
        agg = part_ref[...] / dcol
        o_ref[...] = (
            jnp.dot(x_ref[...], ws_ref[...], preferred_element_type=jnp.float32)
            + jnp.dot(agg, wn_ref[...], preferred_element_type=jnp.float32)
            + b_ref[...]
        )

    return pl.pallas_call(
        body,
        grid=(N // R,),
        in_specs=[
            pl.BlockSpec((R, D), lambda i: (i, 0)),
            pl.BlockSpec((R, D), lambda i: (i, 0)),
            pl.BlockSpec((NC, R, DEGW), lambda i: (0, i, 0)),
            pl.BlockSpec((D, D), lambda i: (0, 0)),
            pl.BlockSpec((D, D), lambda i: (0, 0)),
            pl.BlockSpec((1, D), lambda i: (0, 0)),
        ],
        out_specs=pl.BlockSpec((R, D), lambda i: (i, 0)),
        out_shape=jax.ShapeDtypeStruct((N, D), jnp.float32),
    )(x, part, degp, W_self, W_neigh, bias2d)


def kernel(x, edge_index, W_self, W_neigh, bias):
    ei = edge_index.astype(jnp.int32).reshape(2, NS, NCHUNK, CHUNK)
    x2 = x.reshape(2 * N, HD)
    part, degp = _sc_aggregate(x2, ei)
    return _tc_combine(x, part, degp, W_self, W_neigh, bias.reshape(1, D))
